# Initial kernel scaffold; baseline (speedup 1.0000x reference)
import jax
import jax.numpy as jnp
from jax.experimental import pallas as pl


def kernel(x, edge_index, edge_weight, W_in, b_in, Ws, bs, Wn, bn, ln_g, ln_b, cg, cb, Wc, bc):
    raise NotImplementedError("write your pallas kernel here")


# speedup vs baseline: 1.0160x; 1.0160x over previous
import functools

import jax
import jax.numpy as jnp
from jax.experimental import pallas as pl
from jax.experimental.pallas import tpu as pltpu

N = 10000
E = 160000
D_IN = 256
H = 512
C = 16
L = 3

ROW_BLK = 1000


def _in_proj_body(x_ref, w_ref, b_ref, o_ref):
    o_ref[...] = (
        jnp.dot(x_ref[...], w_ref[...], preferred_element_type=jnp.float32)
        + b_ref[...]
    )


def _in_proj(x, w_t, b):
    return pl.pallas_call(
        _in_proj_body,
        grid=(N // ROW_BLK,),
        in_specs=[
            pl.BlockSpec((ROW_BLK, D_IN), lambda i: (i, 0)),
            pl.BlockSpec((D_IN, H), lambda i: (0, 0)),
            pl.BlockSpec((1, H), lambda i: (0, 0)),
        ],
        out_specs=pl.BlockSpec((ROW_BLK, H), lambda i: (i, 0)),
        out_shape=jax.ShapeDtypeStruct((N, H), jnp.float32),
    )(x, w_t, b)


def _layer_body(h_ref, agg_ref, ws_ref, wn_ref, bs_ref, bn_ref, g_ref, b_ref, o_ref):
    h = h_ref[...]
    h_self = jnp.dot(h, ws_ref[...], preferred_element_type=jnp.float32) + bs_ref[...]
    h_nb = (
        jnp.dot(agg_ref[...], wn_ref[...], preferred_element_type=jnp.float32)
        + bn_ref[...]
    )
    h2 = jnp.maximum(h_self + h_nb, 0.0)
    m = jnp.mean(h2, axis=-1, keepdims=True)
    d = h2 - m
    v = jnp.mean(d * d, axis=-1, keepdims=True)
    h2 = d * jax.lax.rsqrt(v + 1e-5) * g_ref[...] + b_ref[...]
    o_ref[...] = h2 + h


def _layer(h, agg, ws_t, wn_t, bs, bn, g, b):
    return pl.pallas_call(
        _layer_body,
        grid=(N // ROW_BLK,),
        in_specs=[
            pl.BlockSpec((ROW_BLK, H), lambda i: (i, 0)),
            pl.BlockSpec((ROW_BLK, H), lambda i: (i, 0)),
            pl.BlockSpec((H, H), lambda i: (0, 0)),
            pl.BlockSpec((H, H), lambda i: (0, 0)),
            pl.BlockSpec((1, H), lambda i: (0, 0)),
            pl.BlockSpec((1, H), lambda i: (0, 0)),
            pl.BlockSpec((1, H), lambda i: (0, 0)),
            pl.BlockSpec((1, H), lambda i: (0, 0)),
        ],
        out_specs=pl.BlockSpec((ROW_BLK, H), lambda i: (i, 0)),
        out_shape=jax.ShapeDtypeStruct((N, H), jnp.float32),
    )(h, agg, ws_t, wn_t, bs, bn, g, b)


def _final_body(h_ref, g_ref, b_ref, wc_ref, bc_ref, o_ref):
    h = h_ref[...]
    m = jnp.mean(h, axis=-1, keepdims=True)
    d = h - m
    v = jnp.mean(d * d, axis=-1, keepdims=True)
    hc = d * jax.lax.rsqrt(v + 1e-5) * g_ref[...] + b_ref[...]
    o_ref[...] = (
        jnp.dot(hc, wc_ref[...], preferred_element_type=jnp.float32) + bc_ref[...]
    )


def _final(h, g, b, wc_t, bc):
    return pl.pallas_call(
        _final_body,
        grid=(N // ROW_BLK,),
        in_specs=[
            pl.BlockSpec((ROW_BLK, H), lambda i: (i, 0)),
            pl.BlockSpec((1, H), lambda i: (0, 0)),
            pl.BlockSpec((1, H), lambda i: (0, 0)),
            pl.BlockSpec((H, C), lambda i: (0, 0)),
            pl.BlockSpec((1, C), lambda i: (0, 0)),
        ],
        out_specs=pl.BlockSpec((ROW_BLK, C), lambda i: (i, 0)),
        out_shape=jax.ShapeDtypeStruct((N, C), jnp.float32),
    )(h, g, b, wc_t, bc)


def kernel(x, edge_index, edge_weight, W_in, b_in, Ws, bs, Wn, bn, ln_g, ln_b, cg, cb, Wc, bc):
    src = edge_index[0]
    dst = edge_index[1]
    h = _in_proj(x, W_in.T, b_in[None, :])
    for i in range(L):
        msgs = h[src] * edge_weight[:, None]
        agg = jnp.zeros((N, H), dtype=h.dtype).at[dst].add(msgs)
        h = _layer(
            h, agg, Ws[i].T, Wn[i].T,
            bs[i][None, :], bn[i][None, :], ln_g[i][None, :], ln_b[i][None, :],
        )
    return _final(h, cg[None, :], cb[None, :], Wc.T, bc[None, :])


# speedup vs baseline: 3.4768x; 3.4221x over previous
import functools

import jax
import jax.numpy as jnp
from jax import lax
from jax.experimental import pallas as pl
from jax.experimental.pallas import tpu as pltpu
from jax.experimental.pallas import tpu_sc as plsc

N = 10000
E = 160000
D_IN = 256
H = 512
C = 16
L = 3

ROW_BLK = 1000

NSLICE = 4
SLICE_W = 128
NSUB = 16
EPT = E // NSUB
K = 80
NCH = EPT // K
WR_TILES = 10
WR_ROWS = N // WR_TILES


def _in_proj_body(x_ref, w_ref, b_ref, o_ref, ocs_ref):
    h = (
        jnp.dot(x_ref[...], w_ref[...], preferred_element_type=jnp.float32)
        + b_ref[...]
    )
    o_ref[...] = h
    for c in range(NSLICE):
        ocs_ref[c] = h[:, c * SLICE_W:(c + 1) * SLICE_W]


def _in_proj(x, w_t, b):
    return pl.pallas_call(
        _in_proj_body,
        grid=(N // ROW_BLK,),
        in_specs=[
            pl.BlockSpec((ROW_BLK, D_IN), lambda i: (i, 0)),
            pl.BlockSpec((D_IN, H), lambda i: (0, 0)),
            pl.BlockSpec((1, H), lambda i: (0, 0)),
        ],
        out_specs=[
            pl.BlockSpec((ROW_BLK, H), lambda i: (i, 0)),
            pl.BlockSpec((NSLICE, ROW_BLK, SLICE_W), lambda i: (0, i, 0)),
        ],
        out_shape=[
            jax.ShapeDtypeStruct((N, H), jnp.float32),
            jax.ShapeDtypeStruct((NSLICE, N, SLICE_W), jnp.float32),
        ],
    )(x, w_t, b)


def _layer_body(h_ref, agg_ref, ws_ref, wn_ref, bs_ref, bn_ref, g_ref, b_ref,
                o_ref, ocs_ref):
    h = h_ref[...]
    agg = jnp.concatenate([agg_ref[c] for c in range(NSLICE)], axis=-1)
    h_self = jnp.dot(h, ws_ref[...], preferred_element_type=jnp.float32) + bs_ref[...]
    h_nb = jnp.dot(agg, wn_ref[...], preferred_element_type=jnp.float32) + bn_ref[...]
    h2 = jnp.maximum(h_self + h_nb, 0.0)
    m = jnp.mean(h2, axis=-1, keepdims=True)
    d = h2 - m
    v = jnp.mean(d * d, axis=-1, keepdims=True)
    h2 = d * lax.rsqrt(v + 1e-5) * g_ref[...] + b_ref[...]
    hn = h2 + h
    o_ref[...] = hn
    for c in range(NSLICE):
        ocs_ref[c] = hn[:, c * SLICE_W:(c + 1) * SLICE_W]


def _layer(h, agg4, ws_t, wn_t, bs, bn, g, b):
    return pl.pallas_call(
        _layer_body,
        grid=(N // ROW_BLK,),
        in_specs=[
            pl.BlockSpec((ROW_BLK, H), lambda i: (i, 0)),
            pl.BlockSpec((NSLICE, ROW_BLK, SLICE_W), lambda i: (0, i, 0)),
            pl.BlockSpec((H, H), lambda i: (0, 0)),
            pl.BlockSpec((H, H), lambda i: (0, 0)),
            pl.BlockSpec((1, H), lambda i: (0, 0)),
            pl.BlockSpec((1, H), lambda i: (0, 0)),
            pl.BlockSpec((1, H), lambda i: (0, 0)),
            pl.BlockSpec((1, H), lambda i: (0, 0)),
        ],
        out_specs=[
            pl.BlockSpec((ROW_BLK, H), lambda i: (i, 0)),
            pl.BlockSpec((NSLICE, ROW_BLK, SLICE_W), lambda i: (0, i, 0)),
        ],
        out_shape=[
            jax.ShapeDtypeStruct((N, H), jnp.float32),
            jax.ShapeDtypeStruct((NSLICE, N, SLICE_W), jnp.float32),
        ],
    )(h, agg4, ws_t, wn_t, bs, bn, g, b)


def _final_body(h_ref, g_ref, b_ref, wc_ref, bc_ref, o_ref):
    h = h_ref[...]
    m = jnp.mean(h, axis=-1, keepdims=True)
    d = h - m
    v = jnp.mean(d * d, axis=-1, keepdims=True)
    hc = d * lax.rsqrt(v + 1e-5) * g_ref[...] + b_ref[...]
    o_ref[...] = (
        jnp.dot(hc, wc_ref[...], preferred_element_type=jnp.float32) + bc_ref[...]
    )


def _final(h, g, b, wc_t, bc):
    return pl.pallas_call(
        _final_body,
        grid=(N // ROW_BLK,),
        in_specs=[
            pl.BlockSpec((ROW_BLK, H), lambda i: (i, 0)),
            pl.BlockSpec((1, H), lambda i: (0, 0)),
            pl.BlockSpec((1, H), lambda i: (0, 0)),
            pl.BlockSpec((H, C), lambda i: (0, 0)),
            pl.BlockSpec((1, C), lambda i: (0, 0)),
        ],
        out_specs=pl.BlockSpec((ROW_BLK, C), lambda i: (i, 0)),
        out_shape=jax.ShapeDtypeStruct((N, C), jnp.float32),
    )(h, g, b, wc_t, bc)


_SC_MESH = plsc.VectorSubcoreMesh(core_axis_name="c", subcore_axis_name="s")


def _sc_agg_body(hcs, edges, w3, zeros, out, ebuf, wbuf, rows_v, acc,
                 sem_e, sem_w, sem_g):
    cid = lax.axis_index("c")
    sid = lax.axis_index("s")
    for p in range(NSLICE // 2):
        slice_id = cid * (NSLICE // 2) + p

        @pl.when(sid < WR_TILES)
        def _zero():
            pltpu.sync_copy(zeros.at[pl.ds(sid * WR_ROWS, WR_ROWS)],
                            acc.at[pl.ds(sid * WR_ROWS, WR_ROWS)])

        pltpu.async_copy(edges.at[slice_id, sid, 0], ebuf.at[0], sem_e)
        pltpu.async_copy(w3.at[sid, 0], wbuf.at[0], sem_w)
        plsc.subcore_barrier()

        def body(ch, carry):
            par = lax.rem(ch, 2)
            pltpu.make_async_copy(edges.at[slice_id, sid, ch],
                                  ebuf.at[par], sem_e).wait()
            pltpu.make_async_copy(w3.at[sid, ch], wbuf.at[par], sem_w).wait()

            @pl.when(ch + 1 < NCH)
            def _prefetch():
                pltpu.async_copy(edges.at[slice_id, sid, ch + 1],
                                 ebuf.at[1 - par], sem_e)
                pltpu.async_copy(w3.at[sid, ch + 1], wbuf.at[1 - par], sem_w)

            pltpu.async_copy(hcs.at[ebuf.at[par, 0]], rows_v, sem_g).wait()

            def sbody(g, c2):
                base = g * 16
                w16 = wbuf[par, pl.ds(base, 16)]
                for e in range(16):
                    wsp = jnp.full((16,), w16[e], jnp.float32)
                    for c8 in range(8):
                        sl = pl.ds(c8 * 16, 16)
                        rows_v[base + e, sl] = rows_v[base + e, sl] * wsp
                return c2

            lax.fori_loop(0, K // 16, sbody, 0)
            pltpu.sync_copy(rows_v, acc.at[ebuf.at[par, 1]], add=True)
            return carry

        lax.fori_loop(0, NCH, body, 0)
        plsc.subcore_barrier()

        @pl.when(sid < WR_TILES)
        def _writeout():
            pltpu.sync_copy(acc.at[pl.ds(sid * WR_ROWS, WR_ROWS)],
                            out.at[pl.ds(slice_id * N + sid * WR_ROWS, WR_ROWS)])

        plsc.subcore_barrier()


_sc_agg_kernel = functools.partial(
    pl.kernel,
    out_type=jax.ShapeDtypeStruct((NSLICE * N, SLICE_W), jnp.float32),
    mesh=_SC_MESH,
    scratch_types=[
        pltpu.VMEM((2, 2, K), jnp.int32),
        pltpu.VMEM((2, K), jnp.float32),
        pltpu.VMEM((K, SLICE_W), jnp.float32),
        pltpu.VMEM_SHARED((N, SLICE_W), jnp.float32),
        pltpu.SemaphoreType.DMA,
        pltpu.SemaphoreType.DMA,
        pltpu.SemaphoreType.DMA,
    ],
)(_sc_agg_body)


def kernel(x, edge_index, edge_weight, W_in, b_in, Ws, bs, Wn, bn, ln_g, ln_b, cg, cb, Wc, bc):
    src = edge_index[0]
    dst = edge_index[1]
    src3 = src.reshape(NSUB, NCH, K)
    dst3 = dst.reshape(NSUB, NCH, K)
    w3 = edge_weight.reshape(NSUB, NCH, K)
    srcs4 = (src3[None]
             + (jnp.arange(NSLICE, dtype=jnp.int32) * N)[:, None, None, None])
    dst4 = jnp.broadcast_to(dst3[None], (NSLICE,) + dst3.shape)
    edges = jnp.stack([srcs4, dst4], axis=3)
    zeros = jnp.zeros((N, SLICE_W), jnp.float32)

    h, hcs = _in_proj(x, W_in.T, b_in[None, :])
    for i in range(L):
        agg = _sc_agg_kernel(hcs.reshape(NSLICE * N, SLICE_W), edges, w3, zeros)
        agg4 = agg.reshape(NSLICE, N, SLICE_W)
        h, hcs = _layer(
            h, agg4, Ws[i].T, Wn[i].T,
            bs[i][None, :], bn[i][None, :], ln_g[i][None, :], ln_b[i][None, :],
        )
    return _final(h, cg[None, :], cb[None, :], Wc.T, bc[None, :])


# speedup vs baseline: 5.3320x; 1.5336x over previous
import functools

import jax
import jax.numpy as jnp
from jax import lax
from jax.experimental import pallas as pl
from jax.experimental.pallas import tpu as pltpu
from jax.experimental.pallas import tpu_sc as plsc

N = 10000
E = 160000
D_IN = 256
H = 512
C = 16
L = 3

ROW_BLK = 1000

NSLICE = 4
SLICE_W = 128
NSUB = 16
EPT = E // NSUB
K = 80
NCH = EPT // K
WR_TILES = 10
WR_ROWS = N // WR_TILES


def _in_proj_body(x_ref, w_ref, b_ref, o_ref, ocs_ref):
    h = (
        jnp.dot(x_ref[...], w_ref[...], preferred_element_type=jnp.float32)
        + b_ref[...]
    )
    o_ref[...] = h
    for c in range(NSLICE):
        ocs_ref[c] = h[:, c * SLICE_W:(c + 1) * SLICE_W]


def _in_proj(x, w_t, b):
    return pl.pallas_call(
        _in_proj_body,
        grid=(N // ROW_BLK,),
        in_specs=[
            pl.BlockSpec((ROW_BLK, D_IN), lambda i: (i, 0)),
            pl.BlockSpec((D_IN, H), lambda i: (0, 0)),
            pl.BlockSpec((1, H), lambda i: (0, 0)),
        ],
        out_specs=[
            pl.BlockSpec((ROW_BLK, H), lambda i: (i, 0)),
            pl.BlockSpec((NSLICE, ROW_BLK, SLICE_W), lambda i: (0, i, 0)),
        ],
        out_shape=[
            jax.ShapeDtypeStruct((N, H), jnp.float32),
            jax.ShapeDtypeStruct((NSLICE, N, SLICE_W), jnp.float32),
        ],
    )(x, w_t, b)


def _layer_body(h_ref, agg_ref, ws_ref, wn_ref, bs_ref, bn_ref, g_ref, b_ref,
                o_ref, ocs_ref):
    h = h_ref[...]
    agg = jnp.concatenate([agg_ref[c] for c in range(NSLICE)], axis=-1)
    h_self = jnp.dot(h, ws_ref[...], preferred_element_type=jnp.float32) + bs_ref[...]
    h_nb = jnp.dot(agg, wn_ref[...], preferred_element_type=jnp.float32) + bn_ref[...]
    h2 = jnp.maximum(h_self + h_nb, 0.0)
    m = jnp.mean(h2, axis=-1, keepdims=True)
    d = h2 - m
    v = jnp.mean(d * d, axis=-1, keepdims=True)
    h2 = d * lax.rsqrt(v + 1e-5) * g_ref[...] + b_ref[...]
    hn = h2 + h
    o_ref[...] = hn
    for c in range(NSLICE):
        ocs_ref[c] = hn[:, c * SLICE_W:(c + 1) * SLICE_W]


def _layer(h, agg4, ws_t, wn_t, bs, bn, g, b):
    return pl.pallas_call(
        _layer_body,
        grid=(N // ROW_BLK,),
        in_specs=[
            pl.BlockSpec((ROW_BLK, H), lambda i: (i, 0)),
            pl.BlockSpec((NSLICE, ROW_BLK, SLICE_W), lambda i: (0, i, 0)),
            pl.BlockSpec((H, H), lambda i: (0, 0)),
            pl.BlockSpec((H, H), lambda i: (0, 0)),
            pl.BlockSpec((1, H), lambda i: (0, 0)),
            pl.BlockSpec((1, H), lambda i: (0, 0)),
            pl.BlockSpec((1, H), lambda i: (0, 0)),
            pl.BlockSpec((1, H), lambda i: (0, 0)),
        ],
        out_specs=[
            pl.BlockSpec((ROW_BLK, H), lambda i: (i, 0)),
            pl.BlockSpec((NSLICE, ROW_BLK, SLICE_W), lambda i: (0, i, 0)),
        ],
        out_shape=[
            jax.ShapeDtypeStruct((N, H), jnp.float32),
            jax.ShapeDtypeStruct((NSLICE, N, SLICE_W), jnp.float32),
        ],
    )(h, agg4, ws_t, wn_t, bs, bn, g, b)


def _final_body(h_ref, g_ref, b_ref, wc_ref, bc_ref, o_ref):
    h = h_ref[...]
    m = jnp.mean(h, axis=-1, keepdims=True)
    d = h - m
    v = jnp.mean(d * d, axis=-1, keepdims=True)
    hc = d * lax.rsqrt(v + 1e-5) * g_ref[...] + b_ref[...]
    o_ref[...] = (
        jnp.dot(hc, wc_ref[...], preferred_element_type=jnp.float32) + bc_ref[...]
    )


def _final(h, g, b, wc_t, bc):
    return pl.pallas_call(
        _final_body,
        grid=(N // ROW_BLK,),
        in_specs=[
            pl.BlockSpec((ROW_BLK, H), lambda i: (i, 0)),
            pl.BlockSpec((1, H), lambda i: (0, 0)),
            pl.BlockSpec((1, H), lambda i: (0, 0)),
            pl.BlockSpec((H, C), lambda i: (0, 0)),
            pl.BlockSpec((1, C), lambda i: (0, 0)),
        ],
        out_specs=pl.BlockSpec((ROW_BLK, C), lambda i: (i, 0)),
        out_shape=jax.ShapeDtypeStruct((N, C), jnp.float32),
    )(h, g, b, wc_t, bc)


_SC_MESH = plsc.VectorSubcoreMesh(core_axis_name="c", subcore_axis_name="s")


def _sc_agg_body(hcs, edges, w3, zeros, out, ebuf, wbuf, dstbuf, rows_v, acc,
                 sem_e0, sem_e1, sem_w0, sem_w1, sem_g0, sem_g1,
                 sem_s0, sem_s1):
    cid = lax.axis_index("c")
    sid = lax.axis_index("s")
    sem_e = (sem_e0, sem_e1)
    sem_w = (sem_w0, sem_w1)
    sem_g = (sem_g0, sem_g1)
    sem_s = (sem_s0, sem_s1)

    for p in range(NSLICE // 2):
        slice_id = cid * (NSLICE // 2) + p

        def issue_meta(c, s):
            pltpu.async_copy(edges.at[slice_id, sid, c], ebuf.at[s], sem_e[s])
            pltpu.async_copy(w3.at[sid, c], wbuf.at[s], sem_w[s])

        def wait_meta(c, s):
            pltpu.make_async_copy(edges.at[slice_id, sid, c],
                                  ebuf.at[s], sem_e[s]).wait()
            pltpu.make_async_copy(w3.at[sid, c], wbuf.at[s], sem_w[s]).wait()

        def issue_gather(s):
            pltpu.async_copy(hcs.at[ebuf.at[s, 0]], rows_v.at[s], sem_g[s])

        def wait_gather(s):
            pltpu.make_async_copy(hcs.at[ebuf.at[s, 0]], rows_v.at[s],
                                  sem_g[s]).wait()

        def issue_scatter(s):
            pltpu.async_copy(rows_v.at[s], acc.at[dstbuf.at[s]], sem_s[s],
                             add=True)

        def wait_scatter(s):
            pltpu.make_async_copy(rows_v.at[s], acc.at[dstbuf.at[s]],
                                  sem_s[s]).wait()

        def scale_and_stage(s):
            for j in range(K // 16):
                sl = pl.ds(j * 16, 16)
                dstbuf[s, sl] = ebuf[s, 1, sl]

            def sbody(g, c2):
                base = g * 16
                w16 = wbuf[s, pl.ds(base, 16)]
                for e in range(16):
                    wsp = jnp.full((16,), w16[e], jnp.float32)
                    for c8 in range(8):
                        sl2 = pl.ds(c8 * 16, 16)
                        rows_v[s, base + e, sl2] = rows_v[s, base + e, sl2] * wsp
                return c2

            lax.fori_loop(0, K // 16, sbody, 0)

        @pl.when(sid < WR_TILES)
        def _zero():
            pltpu.sync_copy(zeros.at[pl.ds(sid * WR_ROWS, WR_ROWS)],
                            acc.at[pl.ds(sid * WR_ROWS, WR_ROWS)])

        issue_meta(0, 0)
        issue_meta(1, 1)
        wait_meta(0, 0)
        issue_gather(0)
        wait_meta(1, 1)
        issue_gather(1)
        plsc.subcore_barrier()

        def pair(g, carry):
            a = 2 * g
            b = a + 1
            for c, s in ((a, 0), (b, 1)):
                wait_gather(s)
                scale_and_stage(s)
                issue_scatter(s)

                @pl.when(c + 2 < NCH)
                def _next_meta():
                    issue_meta(c + 2, s)

            for c, s in ((a, 0), (b, 1)):
                @pl.when(c + 2 < NCH)
                def _next_gather():
                    wait_scatter(s)
                    wait_meta(c + 2, s)
                    issue_gather(s)

            return carry

        lax.fori_loop(0, NCH // 2, pair, 0)
        wait_gather(0)
        scale_and_stage(0)
        issue_scatter(0)
        wait_scatter(1)
        wait_scatter(0)
        plsc.subcore_barrier()

        @pl.when(sid < WR_TILES)
        def _writeout():
            pltpu.sync_copy(acc.at[pl.ds(sid * WR_ROWS, WR_ROWS)],
                            out.at[pl.ds(slice_id * N + sid * WR_ROWS, WR_ROWS)])

        plsc.subcore_barrier()


_sc_agg_kernel = functools.partial(
    pl.kernel,
    out_type=jax.ShapeDtypeStruct((NSLICE * N, SLICE_W), jnp.float32),
    mesh=_SC_MESH,
    scratch_types=[
        pltpu.VMEM((2, 2, K), jnp.int32),
        pltpu.VMEM((2, K), jnp.float32),
        pltpu.VMEM((2, K), jnp.int32),
        pltpu.VMEM((2, K, SLICE_W), jnp.float32),
        pltpu.VMEM_SHARED((N, SLICE_W), jnp.float32),
        pltpu.SemaphoreType.DMA,
        pltpu.SemaphoreType.DMA,
        pltpu.SemaphoreType.DMA,
        pltpu.SemaphoreType.DMA,
        pltpu.SemaphoreType.DMA,
        pltpu.SemaphoreType.DMA,
        pltpu.SemaphoreType.DMA,
        pltpu.SemaphoreType.DMA,
    ],
)(_sc_agg_body)


def kernel(x, edge_index, edge_weight, W_in, b_in, Ws, bs, Wn, bn, ln_g, ln_b, cg, cb, Wc, bc):
    src = edge_index[0]
    dst = edge_index[1]
    src3 = src.reshape(NSUB, NCH, K)
    dst3 = dst.reshape(NSUB, NCH, K)
    w3 = edge_weight.reshape(NSUB, NCH, K)
    srcs4 = (src3[None]
             + (jnp.arange(NSLICE, dtype=jnp.int32) * N)[:, None, None, None])
    dst4 = jnp.broadcast_to(dst3[None], (NSLICE,) + dst3.shape)
    edges = jnp.stack([srcs4, dst4], axis=3)
    zeros = jnp.zeros((N, SLICE_W), jnp.float32)

    h, hcs = _in_proj(x, W_in.T, b_in[None, :])
    for i in range(L):
        agg = _sc_agg_kernel(hcs.reshape(NSLICE * N, SLICE_W), edges, w3, zeros)
        agg4 = agg.reshape(NSLICE, N, SLICE_W)
        h, hcs = _layer(
            h, agg4, Ws[i].T, Wn[i].T,
            bs[i][None, :], bn[i][None, :], ln_g[i][None, :], ln_b[i][None, :],
        )
    return _final(h, cg[None, :], cb[None, :], Wc.T, bc[None, :])


# speedup vs baseline: 5.8300x; 1.0934x over previous
import functools

import jax
import jax.numpy as jnp
from jax import lax
from jax.experimental import pallas as pl
from jax.experimental.pallas import tpu as pltpu
from jax.experimental.pallas import tpu_sc as plsc

N = 10000
E = 160000
D_IN = 256
H = 512
C = 16
L = 3

ROW_BLK = 1000

NSLICE = 4
SLICE_W = 128
NSUB = 16
EPT = E // NSUB
K = 80
NCH = EPT // K
WR_TILES = 10
WR_ROWS = N // WR_TILES


def _in_proj_body(x_ref, w_ref, b_ref, o_ref, ocs_ref):
    h = (
        jnp.dot(x_ref[...], w_ref[...], preferred_element_type=jnp.float32)
        + b_ref[...]
    )
    o_ref[...] = h
    for c in range(NSLICE):
        ocs_ref[c] = h[:, c * SLICE_W:(c + 1) * SLICE_W]


def _in_proj(x, w_t, b):
    return pl.pallas_call(
        _in_proj_body,
        grid=(N // ROW_BLK,),
        in_specs=[
            pl.BlockSpec((ROW_BLK, D_IN), lambda i: (i, 0)),
            pl.BlockSpec((D_IN, H), lambda i: (0, 0)),
            pl.BlockSpec((1, H), lambda i: (0, 0)),
        ],
        out_specs=[
            pl.BlockSpec((ROW_BLK, H), lambda i: (i, 0)),
            pl.BlockSpec((NSLICE, ROW_BLK, SLICE_W), lambda i: (0, i, 0)),
        ],
        out_shape=[
            jax.ShapeDtypeStruct((N, H), jnp.float32),
            jax.ShapeDtypeStruct((NSLICE, N, SLICE_W), jnp.float32),
        ],
    )(x, w_t, b)


def _layer_body(h_ref, agg_ref, ws_ref, wn_ref, bs_ref, bn_ref, g_ref, b_ref,
                o_ref, ocs_ref):
    h = h_ref[...]
    agg = jnp.concatenate([agg_ref[c] for c in range(NSLICE)], axis=-1)
    h_self = jnp.dot(h, ws_ref[...], preferred_element_type=jnp.float32) + bs_ref[...]
    h_nb = jnp.dot(agg, wn_ref[...], preferred_element_type=jnp.float32) + bn_ref[...]
    h2 = jnp.maximum(h_self + h_nb, 0.0)
    m = jnp.mean(h2, axis=-1, keepdims=True)
    d = h2 - m
    v = jnp.mean(d * d, axis=-1, keepdims=True)
    h2 = d * lax.rsqrt(v + 1e-5) * g_ref[...] + b_ref[...]
    hn = h2 + h
    o_ref[...] = hn
    for c in range(NSLICE):
        ocs_ref[c] = hn[:, c * SLICE_W:(c + 1) * SLICE_W]


def _layer(h, agg4, ws_t, wn_t, bs, bn, g, b):
    return pl.pallas_call(
        _layer_body,
        grid=(N // ROW_BLK,),
        in_specs=[
            pl.BlockSpec((ROW_BLK, H), lambda i: (i, 0)),
            pl.BlockSpec((NSLICE, ROW_BLK, SLICE_W), lambda i: (0, i, 0)),
            pl.BlockSpec((H, H), lambda i: (0, 0)),
            pl.BlockSpec((H, H), lambda i: (0, 0)),
            pl.BlockSpec((1, H), lambda i: (0, 0)),
            pl.BlockSpec((1, H), lambda i: (0, 0)),
            pl.BlockSpec((1, H), lambda i: (0, 0)),
            pl.BlockSpec((1, H), lambda i: (0, 0)),
        ],
        out_specs=[
            pl.BlockSpec((ROW_BLK, H), lambda i: (i, 0)),
            pl.BlockSpec((NSLICE, ROW_BLK, SLICE_W), lambda i: (0, i, 0)),
        ],
        out_shape=[
            jax.ShapeDtypeStruct((N, H), jnp.float32),
            jax.ShapeDtypeStruct((NSLICE, N, SLICE_W), jnp.float32),
        ],
    )(h, agg4, ws_t, wn_t, bs, bn, g, b)


def _final_body(h_ref, g_ref, b_ref, wc_ref, bc_ref, o_ref):
    h = h_ref[...]
    m = jnp.mean(h, axis=-1, keepdims=True)
    d = h - m
    v = jnp.mean(d * d, axis=-1, keepdims=True)
    hc = d * lax.rsqrt(v + 1e-5) * g_ref[...] + b_ref[...]
    o_ref[...] = (
        jnp.dot(hc, wc_ref[...], preferred_element_type=jnp.float32) + bc_ref[...]
    )


def _final(h, g, b, wc_t, bc):
    return pl.pallas_call(
        _final_body,
        grid=(N // ROW_BLK,),
        in_specs=[
            pl.BlockSpec((ROW_BLK, H), lambda i: (i, 0)),
            pl.BlockSpec((1, H), lambda i: (0, 0)),
            pl.BlockSpec((1, H), lambda i: (0, 0)),
            pl.BlockSpec((H, C), lambda i: (0, 0)),
            pl.BlockSpec((1, C), lambda i: (0, 0)),
        ],
        out_specs=pl.BlockSpec((ROW_BLK, C), lambda i: (i, 0)),
        out_shape=jax.ShapeDtypeStruct((N, C), jnp.float32),
    )(h, g, b, wc_t, bc)


_SC_MESH = plsc.VectorSubcoreMesh(core_axis_name="c", subcore_axis_name="s")


NSLOT = 3


def _sc_agg_body(hcs, edges, w3, zeros, out, ebuf, wbuf, dstbuf, rows_v, acc,
                 *sems):
    cid = lax.axis_index("c")
    sid = lax.axis_index("s")
    sem_e = sems[0:NSLOT]
    sem_w = sems[NSLOT:2 * NSLOT]
    sem_g = sems[2 * NSLOT:3 * NSLOT]
    sem_s = sems[3 * NSLOT:4 * NSLOT]

    for p in range(NSLICE // 2):
        slice_id = cid * (NSLICE // 2) + p

        def issue_meta(c, s):
            pltpu.async_copy(edges.at[slice_id, sid, c], ebuf.at[s], sem_e[s])
            pltpu.async_copy(w3.at[sid, c], wbuf.at[s], sem_w[s])

        def wait_meta(c, s):
            pltpu.make_async_copy(edges.at[slice_id, sid, c],
                                  ebuf.at[s], sem_e[s]).wait()
            pltpu.make_async_copy(w3.at[sid, c], wbuf.at[s], sem_w[s]).wait()

        def issue_gather(s):
            pltpu.async_copy(hcs.at[ebuf.at[s, 0]], rows_v.at[s], sem_g[s])

        def wait_gather(s):
            pltpu.make_async_copy(hcs.at[ebuf.at[s, 0]], rows_v.at[s],
                                  sem_g[s]).wait()

        def issue_scatter(s):
            pltpu.async_copy(rows_v.at[s], acc.at[dstbuf.at[s]], sem_s[s],
                             add=True)

        def wait_scatter(s):
            pltpu.make_async_copy(rows_v.at[s], acc.at[dstbuf.at[s]],
                                  sem_s[s]).wait()

        def scale_and_stage(s):
            for j in range(K // 16):
                sl = pl.ds(j * 16, 16)
                dstbuf[s, sl] = ebuf[s, 1, sl]

            def sbody(g, c2):
                base = g * 16
                w16 = wbuf[s, pl.ds(base, 16)]
                for e in range(16):
                    wsp = jnp.full((16,), w16[e], jnp.float32)
                    for c8 in range(8):
                        sl2 = pl.ds(c8 * 16, 16)
                        rows_v[s, base + e, sl2] = rows_v[s, base + e, sl2] * wsp
                return c2

            lax.fori_loop(0, K // 16, sbody, 0)

        @pl.when(sid < WR_TILES)
        def _zero():
            pltpu.sync_copy(zeros.at[pl.ds(sid * WR_ROWS, WR_ROWS)],
                            acc.at[pl.ds(sid * WR_ROWS, WR_ROWS)])

        for s in range(NSLOT):
            issue_meta(s, s)
        for s in range(NSLOT):
            wait_meta(s, s)
            issue_gather(s)
        plsc.subcore_barrier()

        def group(g, carry):
            base = NSLOT * g
            for s in range(NSLOT):
                c = base + s
                wait_gather(s)
                scale_and_stage(s)
                issue_scatter(s)

                @pl.when(c + NSLOT < NCH)
                def _next_meta():
                    issue_meta(c + NSLOT, s)

            for s in range(NSLOT):
                c = base + s

                @pl.when(c + NSLOT < NCH)
                def _next_gather():
                    wait_scatter(s)
                    wait_meta(c + NSLOT, s)
                    issue_gather(s)

            return carry

        nfull = NCH // NSLOT
        lax.fori_loop(0, nfull, group, 0)
        for s in range(NCH % NSLOT):
            wait_gather(s)
            scale_and_stage(s)
            issue_scatter(s)
        for s in range(NSLOT):
            wait_scatter(s)
        plsc.subcore_barrier()

        @pl.when(sid < WR_TILES)
        def _writeout():
            pltpu.sync_copy(acc.at[pl.ds(sid * WR_ROWS, WR_ROWS)],
                            out.at[pl.ds(slice_id * N + sid * WR_ROWS, WR_ROWS)])

        plsc.subcore_barrier()


_sc_agg_kernel = functools.partial(
    pl.kernel,
    out_type=jax.ShapeDtypeStruct((NSLICE * N, SLICE_W), jnp.float32),
    mesh=_SC_MESH,
    scratch_types=[
        pltpu.VMEM((NSLOT, 2, K), jnp.int32),
        pltpu.VMEM((NSLOT, K), jnp.float32),
        pltpu.VMEM((NSLOT, K), jnp.int32),
        pltpu.VMEM((NSLOT, K, SLICE_W), jnp.float32),
        pltpu.VMEM_SHARED((N, SLICE_W), jnp.float32),
    ] + [pltpu.SemaphoreType.DMA] * (4 * NSLOT),
)(_sc_agg_body)


def kernel(x, edge_index, edge_weight, W_in, b_in, Ws, bs, Wn, bn, ln_g, ln_b, cg, cb, Wc, bc):
    src = edge_index[0]
    dst = edge_index[1]
    src3 = src.reshape(NSUB, NCH, K)
    dst3 = dst.reshape(NSUB, NCH, K)
    w3 = edge_weight.reshape(NSUB, NCH, K)
    srcs4 = (src3[None]
             + (jnp.arange(NSLICE, dtype=jnp.int32) * N)[:, None, None, None])
    dst4 = jnp.broadcast_to(dst3[None], (NSLICE,) + dst3.shape)
    edges = jnp.stack([srcs4, dst4], axis=3)
    zeros = jnp.zeros((N, SLICE_W), jnp.float32)

    h, hcs = _in_proj(x, W_in.T, b_in[None, :])
    for i in range(L):
        agg = _sc_agg_kernel(hcs.reshape(NSLICE * N, SLICE_W), edges, w3, zeros)
        agg4 = agg.reshape(NSLICE, N, SLICE_W)
        h, hcs = _layer(
            h, agg4, Ws[i].T, Wn[i].T,
            bs[i][None, :], bn[i][None, :], ln_g[i][None, :], ln_b[i][None, :],
        )
    return _final(h, cg[None, :], cb[None, :], Wc.T, bc[None, :])
